# baseline (speedup 1.0000x reference)
import jax
import jax.numpy as jnp
import numpy as np
from jax import lax
from jax.experimental import pallas as pl
from jax.experimental.pallas import tpu as pltpu
from jax.experimental.pallas import tpu_sc as plsc

_FIELD_DIMS = [100000] * 26
_TOTAL = sum(_FIELD_DIMS)
_F = len(_FIELD_DIMS)
_D = 16
_H = 64
_B = 4096

_NC = 2
_NS = 16
_NW = _NC * _NS
_BPW = _B // _NW


def _sc_gather_body(idx_hbm, embt_hbm, lin_hbm, cross_out, lin_out,
                    idx_v, ev, lin_v, cross_v, lin_sum_v, sem):
    wid = lax.axis_index("c") * _NS + lax.axis_index("s")
    base = wid * _BPW

    pltpu.sync_copy(idx_hbm.at[:, pl.ds(base, _BPW)], idx_v)

    def field_copies(f):
        cs = [pltpu.make_async_copy(
            embt_hbm.at[d].at[idx_v.at[f]], ev.at[f].at[d], sem)
            for d in range(_D)]
        cs.append(pltpu.make_async_copy(
            lin_hbm.at[idx_v.at[f]], lin_v.at[f], sem))
        return cs

    def start_body(f, _):
        for c in field_copies(f):
            c.start()
        return _

    def wait_body(f, _):
        for c in field_copies(f):
            c.wait()
        return _

    lax.fori_loop(0, _F, start_body, None)
    lax.fori_loop(0, _F, wait_body, None)

    def cross_body(d, _):
        for g in range(_BPW // 16):
            s = jnp.zeros((16,), jnp.float32)
            q = jnp.zeros((16,), jnp.float32)
            for f in range(_F):
                v = ev[f, d, pl.ds(g * 16, 16)]
                s = s + v
                q = q + v * v
            cross_v[d, pl.ds(g * 16, 16)] = 0.5 * (s * s - q)
        return _

    lax.fori_loop(0, _D, cross_body, None)

    for g in range(_BPW // 16):
        acc = jnp.zeros((16,), jnp.float32)
        for f in range(_F):
            acc = acc + lin_v[f, pl.ds(g * 16, 16)]
        lin_sum_v[pl.ds(g * 16, 16)] = acc

    pltpu.sync_copy(cross_v, cross_out.at[:, pl.ds(base, _BPW)])
    pltpu.sync_copy(lin_sum_v, lin_out.at[pl.ds(base, _BPW)])


@jax.jit
def _sc_gather(idx_t, embt, lin_flat):
    mesh = plsc.VectorSubcoreMesh(core_axis_name="c", subcore_axis_name="s",
                                  num_cores=_NC, num_subcores=_NS)
    return pl.kernel(
        _sc_gather_body,
        out_type=(jax.ShapeDtypeStruct((_D, _B), jnp.float32),
                  jax.ShapeDtypeStruct((_B,), jnp.float32)),
        mesh=mesh,
        scratch_types=(
            pltpu.VMEM((_F, _BPW), jnp.int32),
            pltpu.VMEM((_F, _D, _BPW), jnp.float32),
            pltpu.VMEM((_F, _BPW), jnp.float32),
            pltpu.VMEM((_D, _BPW), jnp.float32),
            pltpu.VMEM((_BPW,), jnp.float32),
            pltpu.SemaphoreType.DMA,
        ),
        compiler_params=pltpu.CompilerParams(use_tc_tiling_on_sc=False),
    )(idx_t, embt, lin_flat)


def _tc_dense_body(cross_ref, lin_ref, g1_ref, b1_ref, w1_ref, c1_ref,
                   g2_ref, b2_ref, w2_ref, c2_ref, out_ref):
    eps = 1e-5
    c = cross_ref[...]
    mean1 = jnp.mean(c, axis=1, keepdims=True)
    var1 = jnp.mean((c - mean1) ** 2, axis=1, keepdims=True)
    h = (c - mean1) * lax.rsqrt(var1 + eps) * g1_ref[...] + b1_ref[...]
    h = lax.dot_general(w1_ref[...], h, (((0,), (0,)), ((), ())),
                        preferred_element_type=jnp.float32) + c1_ref[...]
    mean2 = jnp.mean(h, axis=1, keepdims=True)
    var2 = jnp.mean((h - mean2) ** 2, axis=1, keepdims=True)
    h = (h - mean2) * lax.rsqrt(var2 + eps) * g2_ref[...] + b2_ref[...]
    h = jnp.maximum(h, 0.0)
    deep = lax.dot_general(w2_ref[...], h, (((0,), (0,)), ((), ())),
                           preferred_element_type=jnp.float32)
    logits = deep + c2_ref[...] + lin_ref[...]
    out_ref[...] = jax.nn.sigmoid(logits)


@jax.jit
def _tc_dense(cross, lin2, g1, b1, w1, c1, g2, b2, w2, c2):
    return pl.pallas_call(
        _tc_dense_body,
        out_shape=jax.ShapeDtypeStruct((1, _B), jnp.float32),
    )(cross, lin2, g1, b1, w1, c1, g2, b2, w2, c2)


def kernel(x, emb_table, lin_w, lin_b, bn1_gamma, bn1_beta, mlp_W, mlp_b,
           bn2_gamma, bn2_beta, out_W, out_b):
    offsets = jnp.asarray(
        np.concatenate([[0], np.cumsum(_FIELD_DIMS)[:-1]]).astype(np.int32))
    idx_t = x.T.astype(jnp.int32) + offsets[:, None]

    cross, lin = _sc_gather(idx_t, emb_table.T, lin_w.reshape(_TOTAL))

    out = _tc_dense(
        cross,
        lin.reshape(1, _B),
        bn1_gamma.reshape(_D, 1), bn1_beta.reshape(_D, 1),
        mlp_W, mlp_b.reshape(_H, 1),
        bn2_gamma.reshape(_H, 1), bn2_beta.reshape(_H, 1),
        out_W, (out_b + lin_b).reshape(1, 1),
    )
    return out.reshape(_B)


# speedup vs baseline: 5.5121x; 5.5121x over previous
import jax
import jax.numpy as jnp
import numpy as np
from jax import lax
from jax.experimental import pallas as pl
from jax.experimental.pallas import tpu as pltpu
from jax.experimental.pallas import tpu_sc as plsc

_FIELD_DIMS = [100000] * 26
_TOTAL = sum(_FIELD_DIMS)
_F = len(_FIELD_DIMS)
_D = 16
_H = 64
_B = 4096

_NC = 2
_NS = 16
_NW = _NC * _NS
_BPW = _B // _NW
_RPW = _F * _BPW


def _sc_gather_body(idx_hbm, emb_hbm, lin_hbm, cross_out, lin_out,
                    idx_v, rows_v, lin_v, cross_v, lin_sum_v, sem):
    wid = lax.axis_index("c") * _NS + lax.axis_index("s")
    base = wid * _BPW

    pltpu.sync_copy(idx_hbm.at[:, pl.ds(base, _BPW)], idx_v)

    copies = []
    for f in range(_F):
        copies.append(pltpu.make_async_copy(
            emb_hbm.at[idx_v.at[f]], rows_v.at[pl.ds(f * _BPW, _BPW), :], sem))
        copies.append(pltpu.make_async_copy(
            lin_hbm.at[idx_v.at[f]], lin_v.at[f], sem))
    for c in copies:
        c.start()
    for c in copies:
        c.wait()

    def cross_body(b, _):
        s = jnp.zeros((_D,), jnp.float32)
        q = jnp.zeros((_D,), jnp.float32)
        for f in range(_F):
            r = rows_v[b + f * _BPW, :]
            s = s + r
            q = q + r * r
        cross_v[b, :] = 0.5 * (s * s - q)
        return _

    lax.fori_loop(0, _BPW, cross_body, None)

    for g in range(_BPW // 16):
        acc = jnp.zeros((16,), jnp.float32)
        for f in range(_F):
            acc = acc + lin_v[f, pl.ds(g * 16, 16)]
        lin_sum_v[pl.ds(g * 16, 16)] = acc

    pltpu.sync_copy(cross_v, cross_out.at[pl.ds(base, _BPW)])
    pltpu.sync_copy(lin_sum_v, lin_out.at[pl.ds(base, _BPW)])


@jax.jit
def _sc_gather(idx_t, emb3, lin_flat):
    mesh = plsc.VectorSubcoreMesh(core_axis_name="c", subcore_axis_name="s",
                                  num_cores=_NC, num_subcores=_NS)
    return pl.kernel(
        _sc_gather_body,
        out_type=(jax.ShapeDtypeStruct((_B, _D), jnp.float32),
                  jax.ShapeDtypeStruct((_B,), jnp.float32)),
        mesh=mesh,
        scratch_types=(
            pltpu.VMEM((_F, _BPW), jnp.int32),
            pltpu.VMEM((_RPW, _D), jnp.float32),
            pltpu.VMEM((_F, _BPW), jnp.float32),
            pltpu.VMEM((_BPW, _D), jnp.float32),
            pltpu.VMEM((_BPW,), jnp.float32),
            pltpu.SemaphoreType.DMA,
        ),
        compiler_params=pltpu.CompilerParams(use_tc_tiling_on_sc=False),
    )(idx_t, emb3, lin_flat)


_CW = 1024
_NFULL = _TOTAL // _CW
_CTAIL = _TOTAL - _NFULL * _CW
_NPER = _NFULL // _NW
_NREM = _NFULL - _NPER * _NW


def _sc_transpose_body(embt_hbm, tail_hbm, out_hbm, slab_a, slab_b, col_a,
                       col_b, sem_ia, sem_ib, sem_oa, sem_ob):
    wid = lax.axis_index("c") * _NS + lax.axis_index("s")
    lanes = lax.iota(jnp.int32, 16)

    def c0_of(j):
        return (j * _NW + wid) * _CW

    def start_in(j, slab, sem):
        pltpu.make_async_copy(
            embt_hbm.at[:, pl.ds(c0_of(j), _CW)], slab, sem).start()

    def wait_in(slab, sem):
        pltpu.make_async_copy(
            embt_hbm.at[:, pl.ds(0, _CW)], slab, sem).wait()

    def transpose_chunk(slab, col, cw):
        def col_body(c, _):
            vs = [plsc.load_gather(slab, [lanes, lanes * 0 + c * 8 + u])
                  for u in range(8)]
            for u in range(8):
                col[pl.ds((c * 8 + u) * _D, _D)] = vs[u]
            return _
        lax.fori_loop(0, cw // 8, col_body, None)

    def start_out(j, col, sem):
        pltpu.make_async_copy(
            col, out_hbm.at[pl.ds(c0_of(j) * _D, _CW * _D)], sem).start()

    def wait_out(col, sem):
        pltpu.make_async_copy(
            out_hbm.at[pl.ds(0, _CW * _D)], col, sem).wait()

    start_in(0, slab_a, sem_ia)

    def pair(t, _):
        ja = 2 * t
        jb = 2 * t + 1
        wait_in(slab_a, sem_ia)
        @pl.when(jb < _NPER)
        def _():
            start_in(jb, slab_b, sem_ib)
        @pl.when(t >= 1)
        def _():
            wait_out(col_a, sem_oa)
        transpose_chunk(slab_a, col_a, _CW)
        start_out(ja, col_a, sem_oa)

        @pl.when(jb < _NPER)
        def _():
            wait_in(slab_b, sem_ib)
            @pl.when(jb + 1 < _NPER)
            def _():
                start_in(jb + 1, slab_a, sem_ia)
            @pl.when(t >= 1)
            def _():
                wait_out(col_b, sem_ob)
            transpose_chunk(slab_b, col_b, _CW)
            start_out(jb, col_b, sem_ob)
        return _

    npair = (_NPER + 1) // 2
    lax.fori_loop(0, npair, pair, None)
    wait_out(col_a, sem_oa)
    if _NPER > 1:
        wait_out(col_b, sem_ob)

    @pl.when(wid < _NREM)
    def _():
        base = (_NPER * _NW + wid) * _CW
        pltpu.make_async_copy(
            embt_hbm.at[:, pl.ds(base, _CW)], slab_a, sem_ia).start()
        wait_in(slab_a, sem_ia)
        transpose_chunk(slab_a, col_a, _CW)
        pltpu.sync_copy(col_a, out_hbm.at[pl.ds(base * _D, _CW * _D)])

    if _CTAIL:
        @pl.when(wid == _NW - 1)
        def _():
            pltpu.sync_copy(
                tail_hbm,
                out_hbm.at[pl.ds(_NFULL * _CW * _D, _CTAIL * _D)])


@jax.jit
def _sc_transpose(embt, tail_flat):
    mesh = plsc.VectorSubcoreMesh(core_axis_name="c", subcore_axis_name="s",
                                  num_cores=_NC, num_subcores=_NS)
    return pl.kernel(
        _sc_transpose_body,
        out_type=jax.ShapeDtypeStruct((_TOTAL * _D,), jnp.float32),
        mesh=mesh,
        scratch_types=(
            pltpu.VMEM((_D, _CW), jnp.float32),
            pltpu.VMEM((_D, _CW), jnp.float32),
            pltpu.VMEM((_CW * _D,), jnp.float32),
            pltpu.VMEM((_CW * _D,), jnp.float32),
            pltpu.SemaphoreType.DMA,
            pltpu.SemaphoreType.DMA,
            pltpu.SemaphoreType.DMA,
            pltpu.SemaphoreType.DMA,
        ),
        compiler_params=pltpu.CompilerParams(needs_layout_passes=False),
    )(embt, tail_flat)


def _tc_dense_body(cross_ref, lin_ref, g1_ref, b1_ref, w1_ref, c1_ref,
                   g2_ref, b2_ref, w2_ref, c2_ref, out_ref):
    eps = 1e-5
    c = cross_ref[...]
    mean1 = jnp.mean(c, axis=0, keepdims=True)
    var1 = jnp.mean((c - mean1) ** 2, axis=0, keepdims=True)
    h = (c - mean1) * lax.rsqrt(var1 + eps) * g1_ref[...] + b1_ref[...]
    h = jnp.dot(h, w1_ref[...], preferred_element_type=jnp.float32) + c1_ref[...]
    mean2 = jnp.mean(h, axis=0, keepdims=True)
    var2 = jnp.mean((h - mean2) ** 2, axis=0, keepdims=True)
    h = (h - mean2) * lax.rsqrt(var2 + eps) * g2_ref[...] + b2_ref[...]
    h = jnp.maximum(h, 0.0)
    deep = jnp.dot(h, w2_ref[...], preferred_element_type=jnp.float32)
    logits = deep + c2_ref[...] + lin_ref[...]
    out_ref[...] = jax.nn.sigmoid(logits)


@jax.jit
def _tc_dense(cross, lin2, g1, b1, w1, c1, g2, b2, w2, c2):
    return pl.pallas_call(
        _tc_dense_body,
        out_shape=jax.ShapeDtypeStruct((_B, 1), jnp.float32),
    )(cross, lin2, g1, b1, w1, c1, g2, b2, w2, c2)


def kernel(x, emb_table, lin_w, lin_b, bn1_gamma, bn1_beta, mlp_W, mlp_b,
           bn2_gamma, bn2_beta, out_W, out_b):
    offsets = jnp.asarray(
        np.concatenate([[0], np.cumsum(_FIELD_DIMS)[:-1]]).astype(np.int32))
    idx_t = x.T.astype(jnp.int32) + offsets[:, None]

    tail_flat = emb_table[_NFULL * _CW:, :].reshape(_CTAIL * _D)
    emb_rm = _sc_transpose(emb_table.T, tail_flat).reshape(_TOTAL, _D)
    cross, lin = _sc_gather(idx_t, emb_rm, lin_w.reshape(_TOTAL))

    out = _tc_dense(
        cross,
        lin.reshape(_B, 1),
        bn1_gamma.reshape(1, _D), bn1_beta.reshape(1, _D),
        mlp_W, mlp_b.reshape(1, _H),
        bn2_gamma.reshape(1, _H), bn2_beta.reshape(1, _H),
        out_W, (out_b + lin_b).reshape(1, 1),
    )
    return out.reshape(_B)


# speedup vs baseline: 5.6378x; 1.0228x over previous
import jax
import jax.numpy as jnp
import numpy as np
from jax import lax
from jax.experimental import pallas as pl
from jax.experimental.pallas import tpu as pltpu
from jax.experimental.pallas import tpu_sc as plsc

_FIELD_DIMS = [100000] * 26
_TOTAL = sum(_FIELD_DIMS)
_F = len(_FIELD_DIMS)
_D = 16
_H = 64
_B = 4096

_NC = 2
_NS = 16
_NW = _NC * _NS
_BPW = _B // _NW
_RPW = _F * _BPW


def _sc_gather_body(idx_hbm, emb_hbm, lin_hbm, cross_out, lin_out,
                    idx_v, rows_v, lin_v, cross_v, lin_sum_v, sem):
    wid = lax.axis_index("c") * _NS + lax.axis_index("s")
    base = wid * _BPW

    pltpu.sync_copy(idx_hbm.at[:, pl.ds(base, _BPW)], idx_v)

    copies = []
    for f in range(_F):
        copies.append(pltpu.make_async_copy(
            emb_hbm.at[idx_v.at[f]], rows_v.at[pl.ds(f * _BPW, _BPW), :], sem))
        copies.append(pltpu.make_async_copy(
            lin_hbm.at[idx_v.at[f]], lin_v.at[f], sem))
    for c in copies:
        c.start()
    for c in copies:
        c.wait()

    def cross_body(b, _):
        s = jnp.zeros((_D,), jnp.float32)
        q = jnp.zeros((_D,), jnp.float32)
        for f in range(_F):
            r = rows_v[b + f * _BPW, :]
            s = s + r
            q = q + r * r
        cross_v[b, :] = 0.5 * (s * s - q)
        return _

    lax.fori_loop(0, _BPW, cross_body, None)

    for g in range(_BPW // 16):
        acc = jnp.zeros((16,), jnp.float32)
        for f in range(_F):
            acc = acc + lin_v[f, pl.ds(g * 16, 16)]
        lin_sum_v[pl.ds(g * 16, 16)] = acc

    pltpu.sync_copy(cross_v, cross_out.at[pl.ds(base, _BPW)])
    pltpu.sync_copy(lin_sum_v, lin_out.at[pl.ds(base, _BPW)])


@jax.jit
def _sc_gather(idx_t, emb3, lin_flat):
    mesh = plsc.VectorSubcoreMesh(core_axis_name="c", subcore_axis_name="s",
                                  num_cores=_NC, num_subcores=_NS)
    return pl.kernel(
        _sc_gather_body,
        out_type=(jax.ShapeDtypeStruct((_B, _D), jnp.float32),
                  jax.ShapeDtypeStruct((_B,), jnp.float32)),
        mesh=mesh,
        scratch_types=(
            pltpu.VMEM((_F, _BPW), jnp.int32),
            pltpu.VMEM((_RPW, _D), jnp.float32),
            pltpu.VMEM((_F, _BPW), jnp.float32),
            pltpu.VMEM((_BPW, _D), jnp.float32),
            pltpu.VMEM((_BPW,), jnp.float32),
            pltpu.SemaphoreType.DMA,
        ),
        compiler_params=pltpu.CompilerParams(use_tc_tiling_on_sc=False),
    )(idx_t, emb3, lin_flat)


_CW = 1024
_NFULL = _TOTAL // _CW
_CTAIL = _TOTAL - _NFULL * _CW
_NPER = _NFULL // _NW
_NREM = _NFULL - _NPER * _NW


def _sc_transpose_body(embt_hbm, tail_hbm, out_hbm, slab_a, slab_b, col_a,
                       col_b, sem_ia, sem_ib, sem_oa, sem_ob):
    wid = lax.axis_index("c") * _NS + lax.axis_index("s")
    lanes = lax.iota(jnp.int32, 16)

    def c0_of(j):
        return (j * _NW + wid) * _CW

    def start_in(j, slab, sem):
        pltpu.make_async_copy(
            embt_hbm.at[:, pl.ds(c0_of(j), _CW)], slab, sem).start()

    def wait_in(slab, sem):
        pltpu.make_async_copy(
            embt_hbm.at[:, pl.ds(0, _CW)], slab, sem).wait()

    def transpose_chunk(slab, col, cw):
        def col_body(c, _):
            vs = [plsc.load_gather(slab, [lanes, lanes * 0 + c * 16 + u])
                  for u in range(16)]
            for u in range(16):
                col[pl.ds((c * 16 + u) * _D, _D)] = vs[u]
            return _
        lax.fori_loop(0, cw // 16, col_body, None)

    def start_out(j, col, sem):
        pltpu.make_async_copy(
            col, out_hbm.at[pl.ds(c0_of(j) * _D, _CW * _D)], sem).start()

    def wait_out(col, sem):
        pltpu.make_async_copy(
            out_hbm.at[pl.ds(0, _CW * _D)], col, sem).wait()

    start_in(0, slab_a, sem_ia)

    def pair(t, _):
        ja = 2 * t
        jb = 2 * t + 1
        wait_in(slab_a, sem_ia)
        @pl.when(jb < _NPER)
        def _():
            start_in(jb, slab_b, sem_ib)
        @pl.when(t >= 1)
        def _():
            wait_out(col_a, sem_oa)
        transpose_chunk(slab_a, col_a, _CW)
        start_out(ja, col_a, sem_oa)

        @pl.when(jb < _NPER)
        def _():
            wait_in(slab_b, sem_ib)
            @pl.when(jb + 1 < _NPER)
            def _():
                start_in(jb + 1, slab_a, sem_ia)
            @pl.when(t >= 1)
            def _():
                wait_out(col_b, sem_ob)
            transpose_chunk(slab_b, col_b, _CW)
            start_out(jb, col_b, sem_ob)
        return _

    npair = (_NPER + 1) // 2
    lax.fori_loop(0, npair, pair, None)
    wait_out(col_a, sem_oa)
    if _NPER > 1:
        wait_out(col_b, sem_ob)

    @pl.when(wid < _NREM)
    def _():
        base = (_NPER * _NW + wid) * _CW
        pltpu.make_async_copy(
            embt_hbm.at[:, pl.ds(base, _CW)], slab_a, sem_ia).start()
        wait_in(slab_a, sem_ia)
        transpose_chunk(slab_a, col_a, _CW)
        pltpu.sync_copy(col_a, out_hbm.at[pl.ds(base * _D, _CW * _D)])

    if _CTAIL:
        @pl.when(wid == _NW - 1)
        def _():
            pltpu.sync_copy(
                tail_hbm,
                out_hbm.at[pl.ds(_NFULL * _CW * _D, _CTAIL * _D)])


@jax.jit
def _sc_transpose(embt, tail_flat):
    mesh = plsc.VectorSubcoreMesh(core_axis_name="c", subcore_axis_name="s",
                                  num_cores=_NC, num_subcores=_NS)
    return pl.kernel(
        _sc_transpose_body,
        out_type=jax.ShapeDtypeStruct((_TOTAL * _D,), jnp.float32),
        mesh=mesh,
        scratch_types=(
            pltpu.VMEM((_D, _CW), jnp.float32),
            pltpu.VMEM((_D, _CW), jnp.float32),
            pltpu.VMEM((_CW * _D,), jnp.float32),
            pltpu.VMEM((_CW * _D,), jnp.float32),
            pltpu.SemaphoreType.DMA,
            pltpu.SemaphoreType.DMA,
            pltpu.SemaphoreType.DMA,
            pltpu.SemaphoreType.DMA,
        ),
        compiler_params=pltpu.CompilerParams(needs_layout_passes=False),
    )(embt, tail_flat)


def _tc_dense_body(cross_ref, lin_ref, g1_ref, b1_ref, w1_ref, c1_ref,
                   g2_ref, b2_ref, w2_ref, c2_ref, out_ref):
    eps = 1e-5
    c = cross_ref[...]
    mean1 = jnp.mean(c, axis=0, keepdims=True)
    var1 = jnp.mean((c - mean1) ** 2, axis=0, keepdims=True)
    h = (c - mean1) * lax.rsqrt(var1 + eps) * g1_ref[...] + b1_ref[...]
    h = jnp.dot(h, w1_ref[...], preferred_element_type=jnp.float32) + c1_ref[...]
    mean2 = jnp.mean(h, axis=0, keepdims=True)
    var2 = jnp.mean((h - mean2) ** 2, axis=0, keepdims=True)
    h = (h - mean2) * lax.rsqrt(var2 + eps) * g2_ref[...] + b2_ref[...]
    h = jnp.maximum(h, 0.0)
    deep = jnp.dot(h, w2_ref[...], preferred_element_type=jnp.float32)
    logits = deep + c2_ref[...] + lin_ref[...]
    out_ref[...] = jax.nn.sigmoid(logits)


@jax.jit
def _tc_dense(cross, lin2, g1, b1, w1, c1, g2, b2, w2, c2):
    return pl.pallas_call(
        _tc_dense_body,
        out_shape=jax.ShapeDtypeStruct((_B, 1), jnp.float32),
    )(cross, lin2, g1, b1, w1, c1, g2, b2, w2, c2)


def kernel(x, emb_table, lin_w, lin_b, bn1_gamma, bn1_beta, mlp_W, mlp_b,
           bn2_gamma, bn2_beta, out_W, out_b):
    offsets = jnp.asarray(
        np.concatenate([[0], np.cumsum(_FIELD_DIMS)[:-1]]).astype(np.int32))
    idx_t = x.T.astype(jnp.int32) + offsets[:, None]

    tail_flat = emb_table[_NFULL * _CW:, :].reshape(_CTAIL * _D)
    emb_rm = _sc_transpose(emb_table.T, tail_flat).reshape(_TOTAL, _D)
    cross, lin = _sc_gather(idx_t, emb_rm, lin_w.reshape(_TOTAL))

    out = _tc_dense(
        cross,
        lin.reshape(_B, 1),
        bn1_gamma.reshape(1, _D), bn1_beta.reshape(1, _D),
        mlp_W, mlp_b.reshape(1, _H),
        bn2_gamma.reshape(1, _H), bn2_beta.reshape(1, _H),
        out_W, (out_b + lin_b).reshape(1, 1),
    )
    return out.reshape(_B)
